# Initial kernel scaffold; baseline (speedup 1.0000x reference)
import jax
import jax.numpy as jnp
from jax.experimental import pallas as pl


def kernel(x):
    raise NotImplementedError("write your pallas kernel here")


# speedup vs baseline: 2.0013x; 2.0013x over previous
import jax
import jax.numpy as jnp
import numpy as np
from jax.experimental import pallas as pl
from jax.experimental.pallas import tpu as pltpu

_K = 512


def _stage(x, j, asc):
    n, c = x.shape
    m = n // (2 * j)
    v = x.reshape(m, 2, j, c)
    a, b = v[:, 0], v[:, 1]
    mn = jnp.minimum(a, b)
    mx = jnp.maximum(a, b)
    if asc is True:
        a2, b2 = mn, mx
    elif asc is None:
        a2, b2 = mx, mn
    else:
        g = jax.lax.broadcasted_iota(jnp.int32, (m, 1, c), 0)
        d = ((g * (2 * j)) & asc) == 0
        a2 = jnp.where(d, mn, mx)
        b2 = jnp.where(d, mx, mn)
    return jnp.stack([a2, b2], axis=1).reshape(n, c)


def _sort_asc(x):
    n = x.shape[0]
    k = 2
    while k <= n:
        j = k // 2
        while j >= 1:
            m = n // (2 * j)
            base = np.arange(m) * 2 * j
            asc = (base & k) == 0
            if asc.all():
                ag = True
            elif not asc.any():
                ag = None
            else:
                ag = k
            x = _stage(x, j, ag)
            j //= 2
        k *= 2
    return x


def _merge_desc(x):
    j = x.shape[0] // 2
    while j >= 1:
        x = _stage(x, j, None)
        j //= 2
    return x


def _body(x_ref, o_ref, r_ref):
    lc = pl.program_id(1)

    @pl.when(lc == 0)
    def _():
        r_ref[...] = jnp.full_like(r_ref, -jnp.inf)

    s = _sort_asc(x_ref[0])
    r_ref[...] = _merge_desc(jnp.maximum(r_ref[...], s))

    @pl.when(lc == pl.num_programs(1) - 1)
    def _():
        o_ref[0] = r_ref[...]


def kernel(x):
    b, l, c = x.shape
    nblk = l // _K
    return pl.pallas_call(
        _body,
        grid=(b, nblk),
        in_specs=[pl.BlockSpec((1, _K, c), lambda bi, li: (bi, li, 0))],
        out_specs=pl.BlockSpec((1, _K, c), lambda bi, li: (bi, 0, 0)),
        out_shape=jax.ShapeDtypeStruct((b, _K, c), x.dtype),
        scratch_shapes=[pltpu.VMEM((_K, c), x.dtype)],
        compiler_params=pltpu.CompilerParams(
            dimension_semantics=("parallel", "arbitrary")
        ),
    )(x)


# speedup vs baseline: 8.6815x; 4.3379x over previous
import jax
import jax.numpy as jnp
from jax.experimental import pallas as pl
from jax.experimental.pallas import tpu as pltpu

_K = 512


def _stage(x, j, k):
    n, _ = x.shape
    i = jax.lax.broadcasted_iota(jnp.int32, (n, 1), 0)
    bitj = (i & j) != 0
    up = pltpu.roll(x, n - j, 0)
    dn = pltpu.roll(x, j, 0)
    partner = jnp.where(bitj, dn, up)
    mx = jnp.maximum(x, partner)
    mn = jnp.minimum(x, partner)
    if k is None:
        keepmax = jnp.logical_not(bitj)
    else:
        keepmax = bitj != ((i & k) != 0)
    return jnp.where(keepmax, mx, mn)


def _sort_asc(x):
    n = x.shape[0]
    k = 2
    while k <= n:
        j = k // 2
        while j >= 1:
            x = _stage(x, j, k)
            j //= 2
        k *= 2
    return x


def _merge_desc(x):
    j = x.shape[0] // 2
    while j >= 1:
        x = _stage(x, j, None)
        j //= 2
    return x


def _body(x_ref, o_ref, r_ref):
    lc = pl.program_id(1)

    @pl.when(lc == 0)
    def _():
        r_ref[...] = jnp.full_like(r_ref, -jnp.inf)

    xb = x_ref[...]
    s = jnp.concatenate([xb[0], xb[1]], axis=-1)
    s = _sort_asc(s)
    r_ref[...] = _merge_desc(jnp.maximum(r_ref[...], s))

    @pl.when(lc == pl.num_programs(1) - 1)
    def _():
        c = o_ref.shape[-1]
        o_ref[0] = r_ref[:, :c]
        o_ref[1] = r_ref[:, c:]


def kernel(x):
    b, l, c = x.shape
    nblk = l // _K
    return pl.pallas_call(
        _body,
        grid=(b // 2, nblk),
        in_specs=[pl.BlockSpec((2, _K, c), lambda bi, li: (bi, li, 0))],
        out_specs=pl.BlockSpec((2, _K, c), lambda bi, li: (bi, 0, 0)),
        out_shape=jax.ShapeDtypeStruct((b, _K, c), x.dtype),
        scratch_shapes=[pltpu.VMEM((_K, 2 * c), x.dtype)],
        compiler_params=pltpu.CompilerParams(
            dimension_semantics=("parallel", "arbitrary")
        ),
    )(x)
